# baseline (speedup 1.0000x reference)
import functools

import jax
import jax.numpy as jnp
import numpy as np
from jax.experimental import pallas as pl
from jax.experimental.pallas import tpu as pltpu
from jax.sharding import Mesh, PartitionSpec as P


def _phased_lstm_kernel(x_ref, wih_ref, whh_ref, b_ref, wfc_ref, bfc_ref,
                        out_ref, hall_sc, gx_sc, h_sc, c_sc):
    lyr = pl.program_id(0)
    blk = pl.program_id(1)
    n_lyr = pl.num_programs(0)
    n_blk = pl.num_programs(1)

    Tc, B, H = x_ref.shape
    H4 = 4 * H

    @pl.when(blk == 0)
    def _():
        h_sc[...] = jnp.zeros_like(h_sc)
        c_sc[...] = jnp.zeros_like(c_sc)

    @pl.when(lyr == 0)
    def _():
        xin = x_ref[...].reshape(Tc * B, H)
        g = jnp.dot(xin, wih_ref[0], preferred_element_type=jnp.float32)
        gx_sc[...] = (g + b_ref[0]).reshape(Tc, B, H4)

    @pl.when(lyr != 0)
    def _():
        hin = hall_sc[pl.ds(blk * Tc, Tc)].reshape(Tc * B, H)
        g = jnp.dot(hin, wih_ref[0], preferred_element_type=jnp.float32)
        gx_sc[...] = (g + b_ref[0]).reshape(Tc, B, H4)

    whh = whh_ref[0]

    def one_step(ts, carry):
        h, c = carry
        gates = gx_sc[ts] + jnp.dot(h.astype(jnp.bfloat16), whh,
                                    preferred_element_type=jnp.float32)
        i_g = jax.nn.sigmoid(gates[:, 0 * H:1 * H])
        f_g = jax.nn.sigmoid(gates[:, 1 * H:2 * H])
        g_g = jnp.tanh(gates[:, 2 * H:3 * H])
        o_g = jax.nn.sigmoid(gates[:, 3 * H:4 * H])
        c_new = f_g * c + i_g * g_g
        h_new = o_g * jnp.tanh(c_new)
        hall_sc[blk * Tc + ts] = h_new.astype(jnp.bfloat16)
        return h_new, c_new

    h, c = jax.lax.fori_loop(0, Tc, one_step,
                             (h_sc[...], c_sc[...]), unroll=True)
    h_sc[...] = h
    c_sc[...] = c

    @pl.when(jnp.logical_and(lyr == n_lyr - 1, blk == n_blk - 1))
    def _():
        logits = jnp.dot(h, wfc_ref[...],
                         preferred_element_type=jnp.float32) + bfc_ref[...]
        out_ref[...] = jax.nn.sigmoid(logits).astype(out_ref.dtype)


def _lstm_net(x_tbh, wih, whh, bias, wfc_t, bfc, *, time_block):
    T, B, H = x_tbh.shape
    L = wih.shape[0]
    out_dim = wfc_t.shape[1]

    tb = min(time_block, T)
    while T % tb != 0:
        tb -= 1

    return pl.pallas_call(
        _phased_lstm_kernel,
        out_shape=jax.ShapeDtypeStruct((B, out_dim), jnp.float32),
        grid=(L, T // tb),
        in_specs=[
            pl.BlockSpec((tb, B, H), lambda l, t: ((1 - l) * t, 0, 0)),
            pl.BlockSpec((1, H, 4 * H), lambda l, t: (l, 0, 0)),
            pl.BlockSpec((1, H, 4 * H), lambda l, t: (l, 0, 0)),
            pl.BlockSpec((1, 1, 4 * H), lambda l, t: (l, 0, 0)),
            pl.BlockSpec((H, out_dim), lambda l, t: (0, 0)),
            pl.BlockSpec((1, out_dim), lambda l, t: (0, 0)),
        ],
        out_specs=pl.BlockSpec((B, out_dim), lambda l, t: (0, 0)),
        scratch_shapes=[
            pltpu.VMEM((T, B, H), jnp.bfloat16),
            pltpu.VMEM((tb, B, 4 * H), jnp.float32),
            pltpu.VMEM((B, H), jnp.float32),
            pltpu.VMEM((B, H), jnp.float32),
        ],
        compiler_params=pltpu.CompilerParams(
            dimension_semantics=("arbitrary", "arbitrary"),
            vmem_limit_bytes=100 * 1024 * 1024),
    )(x_tbh, wih, whh, bias, wfc_t, bfc)


def _net_one_shard(embedding, lstm0_w_ih, lstm0_w_hh, bias0,
                   lstm1_w_ih, lstm1_w_hh, bias1, fc_w, fc_b, tokens,
                   time_block):
    H = embedding.shape[1]

    x = jnp.take(embedding.astype(jnp.bfloat16), tokens.T, axis=0)

    wih = jnp.stack([jnp.transpose(lstm0_w_ih),
                     jnp.transpose(lstm1_w_ih)]).astype(jnp.bfloat16)
    whh = jnp.stack([jnp.transpose(lstm0_w_hh),
                     jnp.transpose(lstm1_w_hh)]).astype(jnp.bfloat16)
    bias = jnp.stack([bias0, bias1])

    wfc_t = jnp.transpose(fc_w)
    bfc = fc_b.reshape(1, -1)

    return _lstm_net(x, wih, whh, bias, wfc_t, bfc, time_block=time_block)


@functools.partial(jax.jit, static_argnames=("time_block",))
def _forward(embedding, lstm0_w_ih, lstm0_w_hh, lstm0_b_ih, lstm0_b_hh,
             lstm1_w_ih, lstm1_w_hh, lstm1_b_ih, lstm1_b_hh,
             fc_w, fc_b, tokens, time_block=16):
    H4 = 4 * embedding.shape[1]
    B = tokens.shape[0]
    bias0 = (lstm0_b_ih + lstm0_b_hh).reshape(1, H4)
    bias1 = (lstm1_b_ih + lstm1_b_hh).reshape(1, H4)

    devs = jax.devices()
    nshard = 2 if (len(devs) >= 2 and B % 16 == 0) else 1
    if nshard > 1:
        mesh = Mesh(np.array(devs[:nshard]), ("b",))
        rep = P()
        fn = jax.shard_map(
            functools.partial(_net_one_shard, time_block=time_block),
            mesh=mesh,
            in_specs=(rep, rep, rep, rep, rep, rep, rep, rep, rep,
                      P("b", None)),
            out_specs=P("b", None),
            check_vma=False,
        )
    else:
        fn = functools.partial(_net_one_shard, time_block=time_block)
    out = fn(embedding, lstm0_w_ih, lstm0_w_hh, bias0,
             lstm1_w_ih, lstm1_w_hh, bias1, fc_w, fc_b, tokens)
    return out.reshape(-1, 1)


def kernel(embedding, lstm0_w_ih, lstm0_w_hh, lstm0_b_ih, lstm0_b_hh,
           lstm1_w_ih, lstm1_w_hh, lstm1_b_ih, lstm1_b_hh,
           fc_w, fc_b, tokens):
    return _forward(embedding, lstm0_w_ih, lstm0_w_hh, lstm0_b_ih, lstm0_b_hh,
                    lstm1_w_ih, lstm1_w_hh, lstm1_b_ih, lstm1_b_hh,
                    fc_w, fc_b, tokens)


# speedup vs baseline: 3.2847x; 3.2847x over previous
import functools

import jax
import jax.numpy as jnp
from jax.experimental import pallas as pl
from jax.experimental.pallas import tpu as pltpu


def _phased_lstm_kernel(x_ref, w0ih_ref, w0hh_ref, w1ih_ref, w1hh_ref,
                        b0_ref, b1_ref, wfc_ref, bfc_ref,
                        out_ref, wih_sc, whh_sc, hall_sc, gx_sc, h_sc, c_sc):
    lyr = pl.program_id(0)
    blk = pl.program_id(1)
    n_lyr = pl.num_programs(0)
    n_blk = pl.num_programs(1)

    Tc, B, H = x_ref.shape
    H4 = 4 * H

    @pl.when(blk == 0)
    def _():
        h_sc[...] = jnp.zeros_like(h_sc)
        c_sc[...] = jnp.zeros_like(c_sc)

    @pl.when(jnp.logical_and(lyr == 0, blk == 0))
    def _():
        wih_sc[...] = jnp.transpose(w0ih_ref[...]).astype(jnp.bfloat16)
        whh_sc[...] = jnp.transpose(w0hh_ref[...]).astype(jnp.bfloat16)

    @pl.when(jnp.logical_and(lyr != 0, blk == 0))
    def _():
        wih_sc[...] = jnp.transpose(w1ih_ref[...]).astype(jnp.bfloat16)
        whh_sc[...] = jnp.transpose(w1hh_ref[...]).astype(jnp.bfloat16)

    @pl.when(lyr == 0)
    def _():
        xin = x_ref[...].reshape(Tc * B, H).astype(jnp.bfloat16)
        g = jnp.dot(xin, wih_sc[...], preferred_element_type=jnp.float32)
        gx_sc[...] = (g + b0_ref[...]).reshape(Tc, B, H4)

    @pl.when(lyr != 0)
    def _():
        hin = hall_sc[pl.ds(blk * Tc, Tc)].reshape(Tc * B, H)
        g = jnp.dot(hin, wih_sc[...], preferred_element_type=jnp.float32)
        gx_sc[...] = (g + b1_ref[...]).reshape(Tc, B, H4)

    def one_step(ts, carry):
        h, c = carry
        gates = gx_sc[ts] + jnp.dot(h.astype(jnp.bfloat16), whh_sc[...],
                                    preferred_element_type=jnp.float32)
        i_g = jax.nn.sigmoid(gates[:, 0 * H:1 * H])
        f_g = jax.nn.sigmoid(gates[:, 1 * H:2 * H])
        g_g = jnp.tanh(gates[:, 2 * H:3 * H])
        o_g = jax.nn.sigmoid(gates[:, 3 * H:4 * H])
        c_new = f_g * c + i_g * g_g
        h_new = o_g * jnp.tanh(c_new)
        hall_sc[blk * Tc + ts] = h_new.astype(jnp.bfloat16)
        return h_new, c_new

    h, c = jax.lax.fori_loop(0, Tc, one_step,
                             (h_sc[...], c_sc[...]), unroll=True)
    h_sc[...] = h
    c_sc[...] = c

    @pl.when(jnp.logical_and(lyr == n_lyr - 1, blk == n_blk - 1))
    def _():
        logits = jnp.dot(h, wfc_ref[...],
                         preferred_element_type=jnp.float32) + bfc_ref[...]
        out_ref[...] = jax.nn.sigmoid(logits).astype(out_ref.dtype)


def _lstm_net(x_tbh, w0ih, w0hh, w1ih, w1hh, b0, b1, wfc, bfc, *, time_block):
    T, B, H = x_tbh.shape
    L = 2
    out_dim = wfc.shape[1]

    tb = min(time_block, T)
    while T % tb != 0:
        tb -= 1

    resident = lambda shape: pl.BlockSpec(shape, lambda l, t: tuple(
        0 for _ in shape))

    return pl.pallas_call(
        _phased_lstm_kernel,
        out_shape=jax.ShapeDtypeStruct((B, out_dim), jnp.float32),
        grid=(L, T // tb),
        in_specs=[
            pl.BlockSpec((tb, B, H), lambda l, t: ((1 - l) * t, 0, 0)),
            resident((4 * H, H)),
            resident((4 * H, H)),
            resident((4 * H, H)),
            resident((4 * H, H)),
            resident((1, 4 * H)),
            resident((1, 4 * H)),
            resident((H, out_dim)),
            resident((1, out_dim)),
        ],
        out_specs=pl.BlockSpec((B, out_dim), lambda l, t: (0, 0)),
        scratch_shapes=[
            pltpu.VMEM((H, 4 * H), jnp.bfloat16),
            pltpu.VMEM((H, 4 * H), jnp.bfloat16),
            pltpu.VMEM((T, B, H), jnp.bfloat16),
            pltpu.VMEM((tb, B, 4 * H), jnp.float32),
            pltpu.VMEM((B, H), jnp.float32),
            pltpu.VMEM((B, H), jnp.float32),
        ],
        compiler_params=pltpu.CompilerParams(
            dimension_semantics=("arbitrary", "arbitrary"),
            vmem_limit_bytes=100 * 1024 * 1024),
    )(x_tbh, w0ih, w0hh, w1ih, w1hh, b0, b1, wfc, bfc)


@functools.partial(jax.jit, static_argnames=("time_block",))
def _forward(embedding, lstm0_w_ih, lstm0_w_hh, lstm0_b_ih, lstm0_b_hh,
             lstm1_w_ih, lstm1_w_hh, lstm1_b_ih, lstm1_b_hh,
             fc_w, fc_b, tokens, time_block=16):
    H4 = 4 * embedding.shape[1]
    x = jnp.take(embedding, tokens.T, axis=0)
    b0 = (lstm0_b_ih + lstm0_b_hh).reshape(1, H4)
    b1 = (lstm1_b_ih + lstm1_b_hh).reshape(1, H4)
    out = _lstm_net(x, lstm0_w_ih, lstm0_w_hh, lstm1_w_ih, lstm1_w_hh,
                    b0, b1, jnp.transpose(fc_w), fc_b.reshape(1, -1),
                    time_block=time_block)
    return out.reshape(-1, 1)


def kernel(embedding, lstm0_w_ih, lstm0_w_hh, lstm0_b_ih, lstm0_b_hh,
           lstm1_w_ih, lstm1_w_hh, lstm1_b_ih, lstm1_b_hh,
           fc_w, fc_b, tokens):
    return _forward(embedding, lstm0_w_ih, lstm0_w_hh, lstm0_b_ih, lstm0_b_hh,
                    lstm1_w_ih, lstm1_w_hh, lstm1_b_ih, lstm1_b_hh,
                    fc_w, fc_b, tokens)


# speedup vs baseline: 3.3230x; 1.0116x over previous
import functools

import jax
import jax.numpy as jnp
from jax.experimental import pallas as pl
from jax.experimental.pallas import tpu as pltpu


def _phased_lstm_kernel(x_ref, w0ih_ref, w0hh_ref, w1ih_ref, w1hh_ref,
                        b0_ref, b1_ref, wfc_ref, bfc_ref,
                        out_ref, wih_sc, whh_sc, hall_sc, gx_sc, h_sc, c_sc):
    lyr = pl.program_id(0)
    blk = pl.program_id(1)
    n_lyr = pl.num_programs(0)
    n_blk = pl.num_programs(1)

    Tc, B, H = x_ref.shape
    H4 = 4 * H

    @pl.when(blk == 0)
    def _():
        h_sc[...] = jnp.zeros_like(h_sc)
        c_sc[...] = jnp.zeros_like(c_sc)

    @pl.when(jnp.logical_and(lyr == 0, blk == 0))
    def _():
        wih_sc[...] = jnp.transpose(w0ih_ref[...]).astype(jnp.bfloat16)
        whh_sc[...] = jnp.transpose(w0hh_ref[...]).astype(jnp.bfloat16)

    @pl.when(jnp.logical_and(lyr != 0, blk == 0))
    def _():
        wih_sc[...] = jnp.transpose(w1ih_ref[...]).astype(jnp.bfloat16)
        whh_sc[...] = jnp.transpose(w1hh_ref[...]).astype(jnp.bfloat16)

    @pl.when(lyr == 0)
    def _():
        xin = x_ref[...].reshape(Tc * B, H).astype(jnp.bfloat16)
        g = jnp.dot(xin, wih_sc[...], preferred_element_type=jnp.float32)
        gx_sc[...] = (g + b0_ref[...]).reshape(Tc, B, H4)

    @pl.when(lyr != 0)
    def _():
        hin = hall_sc[pl.ds(blk * Tc, Tc)].reshape(Tc * B, H)
        g = jnp.dot(hin, wih_sc[...], preferred_element_type=jnp.float32)
        gx_sc[...] = (g + b1_ref[...]).reshape(Tc, B, H4)

    def one_step(ts, carry):
        h, c = carry
        gates = gx_sc[ts] + jnp.dot(h.astype(jnp.bfloat16), whh_sc[...],
                                    preferred_element_type=jnp.float32)
        i_g = jax.nn.sigmoid(gates[:, 0 * H:1 * H])
        f_g = jax.nn.sigmoid(gates[:, 1 * H:2 * H])
        g_g = jnp.tanh(gates[:, 2 * H:3 * H])
        o_g = jax.nn.sigmoid(gates[:, 3 * H:4 * H])
        c_new = f_g * c + i_g * g_g
        h_new = o_g * jnp.tanh(c_new)
        hall_sc[blk * Tc + ts] = h_new.astype(jnp.bfloat16)
        return h_new, c_new

    h, c = jax.lax.fori_loop(0, Tc, one_step,
                             (h_sc[...], c_sc[...]), unroll=True)
    h_sc[...] = h
    c_sc[...] = c

    @pl.when(jnp.logical_and(lyr == n_lyr - 1, blk == n_blk - 1))
    def _():
        logits = jnp.dot(h, wfc_ref[...],
                         preferred_element_type=jnp.float32) + bfc_ref[...]
        out_ref[...] = jax.nn.sigmoid(logits).astype(out_ref.dtype)


def _lstm_net(x_tbh, w0ih, w0hh, w1ih, w1hh, b0, b1, wfc, bfc, *, time_block):
    T, B, H = x_tbh.shape
    L = 2
    out_dim = wfc.shape[1]

    tb = min(time_block, T)
    while T % tb != 0:
        tb -= 1

    resident = lambda shape: pl.BlockSpec(shape, lambda l, t: tuple(
        0 for _ in shape))

    return pl.pallas_call(
        _phased_lstm_kernel,
        out_shape=jax.ShapeDtypeStruct((B, out_dim), jnp.float32),
        grid=(L, T // tb),
        in_specs=[
            pl.BlockSpec((tb, B, H), lambda l, t: ((1 - l) * t, 0, 0)),
            resident((4 * H, H)),
            resident((4 * H, H)),
            resident((4 * H, H)),
            resident((4 * H, H)),
            resident((1, 4 * H)),
            resident((1, 4 * H)),
            resident((H, out_dim)),
            resident((1, out_dim)),
        ],
        out_specs=pl.BlockSpec((B, out_dim), lambda l, t: (0, 0)),
        scratch_shapes=[
            pltpu.VMEM((H, 4 * H), jnp.bfloat16),
            pltpu.VMEM((H, 4 * H), jnp.bfloat16),
            pltpu.VMEM((T, B, H), jnp.bfloat16),
            pltpu.VMEM((tb, B, 4 * H), jnp.float32),
            pltpu.VMEM((B, H), jnp.float32),
            pltpu.VMEM((B, H), jnp.float32),
        ],
        compiler_params=pltpu.CompilerParams(
            dimension_semantics=("arbitrary", "arbitrary"),
            vmem_limit_bytes=100 * 1024 * 1024),
    )(x_tbh, w0ih, w0hh, w1ih, w1hh, b0, b1, wfc, bfc)


@functools.partial(jax.jit, static_argnames=("time_block",))
def _forward(embedding, lstm0_w_ih, lstm0_w_hh, lstm0_b_ih, lstm0_b_hh,
             lstm1_w_ih, lstm1_w_hh, lstm1_b_ih, lstm1_b_hh,
             fc_w, fc_b, tokens, time_block=32):
    H4 = 4 * embedding.shape[1]
    x = jnp.take(embedding, tokens.T, axis=0)
    b0 = (lstm0_b_ih + lstm0_b_hh).reshape(1, H4)
    b1 = (lstm1_b_ih + lstm1_b_hh).reshape(1, H4)
    out = _lstm_net(x, lstm0_w_ih, lstm0_w_hh, lstm1_w_ih, lstm1_w_hh,
                    b0, b1, jnp.transpose(fc_w), fc_b.reshape(1, -1),
                    time_block=time_block)
    return out.reshape(-1, 1)


def kernel(embedding, lstm0_w_ih, lstm0_w_hh, lstm0_b_ih, lstm0_b_hh,
           lstm1_w_ih, lstm1_w_hh, lstm1_b_ih, lstm1_b_hh,
           fc_w, fc_b, tokens):
    return _forward(embedding, lstm0_w_ih, lstm0_w_hh, lstm0_b_ih, lstm0_b_hh,
                    lstm1_w_ih, lstm1_w_hh, lstm1_b_ih, lstm1_b_hh,
                    fc_w, fc_b, tokens)


# speedup vs baseline: 3.7013x; 1.1138x over previous
import functools

import jax
import jax.numpy as jnp
from jax.experimental import pallas as pl
from jax.experimental.pallas import tpu as pltpu


def _skewed_lstm_kernel(x_ref, w0ih_ref, w0hh_ref, w1ih_ref, w1hh_ref,
                        b0_ref, b1_ref, wfc_ref, bfc_ref, out_ref,
                        w0i_sc, w0h_sc, w1i_sc, w1h_sc,
                        hall_sc, gx0_sc, gx1_sc,
                        h0_sc, c0_sc, h1_sc, c1_sc):
    k = pl.program_id(0)
    nb = pl.num_programs(0) - 1

    Tc, B, H = x_ref.shape
    H4 = 4 * H

    @pl.when(k == 0)
    def _():
        h0_sc[...] = jnp.zeros_like(h0_sc)
        c0_sc[...] = jnp.zeros_like(c0_sc)
        h1_sc[...] = jnp.zeros_like(h1_sc)
        c1_sc[...] = jnp.zeros_like(c1_sc)
        w0i_sc[...] = jnp.transpose(w0ih_ref[...]).astype(jnp.bfloat16)
        w0h_sc[...] = jnp.transpose(w0hh_ref[...]).astype(jnp.bfloat16)
        w1i_sc[...] = jnp.transpose(w1ih_ref[...]).astype(jnp.bfloat16)
        w1h_sc[...] = jnp.transpose(w1hh_ref[...]).astype(jnp.bfloat16)

    @pl.when(k < nb)
    def _():
        xin = x_ref[...].reshape(Tc * B, H).astype(jnp.bfloat16)
        g = jnp.dot(xin, w0i_sc[...], preferred_element_type=jnp.float32)
        gx0_sc[...] = (g + b0_ref[...]).reshape(Tc, B, H4)

    @pl.when(k > 0)
    def _():
        hin = hall_sc[pl.ds((k - 1) * Tc, Tc)].reshape(Tc * B, H)
        g = jnp.dot(hin, w1i_sc[...], preferred_element_type=jnp.float32)
        gx1_sc[...] = (g + b1_ref[...]).reshape(Tc, B, H4)

    def lstm_step(gates, c):
        i_g = jax.nn.sigmoid(gates[:, 0 * H:1 * H])
        f_g = jax.nn.sigmoid(gates[:, 1 * H:2 * H])
        g_g = jnp.tanh(gates[:, 2 * H:3 * H])
        o_g = jax.nn.sigmoid(gates[:, 3 * H:4 * H])
        c_new = f_g * c + i_g * g_g
        return o_g * jnp.tanh(c_new), c_new

    def l0_step(ts, h, c):
        gates = gx0_sc[ts] + jnp.dot(h.astype(jnp.bfloat16), w0h_sc[...],
                                     preferred_element_type=jnp.float32)
        h_new, c_new = lstm_step(gates, c)
        hall_sc[k * Tc + ts] = h_new.astype(jnp.bfloat16)
        return h_new, c_new

    def l1_step(ts, h, c):
        gates = gx1_sc[ts] + jnp.dot(h.astype(jnp.bfloat16), w1h_sc[...],
                                     preferred_element_type=jnp.float32)
        return lstm_step(gates, c)

    @pl.when(k == 0)
    def _():
        def body(ts, carry):
            h0, c0 = carry
            return l0_step(ts, h0, c0)
        h0, c0 = jax.lax.fori_loop(0, Tc, body,
                                   (h0_sc[...], c0_sc[...]), unroll=True)
        h0_sc[...] = h0
        c0_sc[...] = c0

    @pl.when(jnp.logical_and(k > 0, k < nb))
    def _():
        def body(ts, carry):
            h0, c0, h1, c1 = carry
            h0, c0 = l0_step(ts, h0, c0)
            h1, c1 = l1_step(ts, h1, c1)
            return h0, c0, h1, c1
        h0, c0, h1, c1 = jax.lax.fori_loop(
            0, Tc, body,
            (h0_sc[...], c0_sc[...], h1_sc[...], c1_sc[...]), unroll=True)
        h0_sc[...] = h0
        c0_sc[...] = c0
        h1_sc[...] = h1
        c1_sc[...] = c1

    @pl.when(k == nb)
    def _():
        def body(ts, carry):
            h1, c1 = carry
            return l1_step(ts, h1, c1)
        h1, c1 = jax.lax.fori_loop(0, Tc, body,
                                   (h1_sc[...], c1_sc[...]), unroll=True)
        logits = jnp.dot(h1, wfc_ref[...],
                         preferred_element_type=jnp.float32) + bfc_ref[...]
        out_ref[...] = jax.nn.sigmoid(logits).astype(out_ref.dtype)


def _lstm_net(x_tbh, w0ih, w0hh, w1ih, w1hh, b0, b1, wfc, bfc, *, time_block):
    T, B, H = x_tbh.shape
    out_dim = wfc.shape[1]

    tb = min(time_block, T)
    while T % tb != 0:
        tb -= 1
    nb = T // tb

    resident = lambda shape: pl.BlockSpec(shape, lambda k: tuple(
        0 for _ in shape))

    return pl.pallas_call(
        _skewed_lstm_kernel,
        out_shape=jax.ShapeDtypeStruct((B, out_dim), jnp.float32),
        grid=(nb + 1,),
        in_specs=[
            pl.BlockSpec((tb, B, H), lambda k: (jnp.minimum(k, nb - 1), 0, 0)),
            resident((4 * H, H)),
            resident((4 * H, H)),
            resident((4 * H, H)),
            resident((4 * H, H)),
            resident((1, 4 * H)),
            resident((1, 4 * H)),
            resident((H, out_dim)),
            resident((1, out_dim)),
        ],
        out_specs=pl.BlockSpec((B, out_dim), lambda k: (0, 0)),
        scratch_shapes=[
            pltpu.VMEM((H, 4 * H), jnp.bfloat16),
            pltpu.VMEM((H, 4 * H), jnp.bfloat16),
            pltpu.VMEM((H, 4 * H), jnp.bfloat16),
            pltpu.VMEM((H, 4 * H), jnp.bfloat16),
            pltpu.VMEM((T, B, H), jnp.bfloat16),
            pltpu.VMEM((tb, B, 4 * H), jnp.float32),
            pltpu.VMEM((tb, B, 4 * H), jnp.float32),
            pltpu.VMEM((B, H), jnp.float32),
            pltpu.VMEM((B, H), jnp.float32),
            pltpu.VMEM((B, H), jnp.float32),
            pltpu.VMEM((B, H), jnp.float32),
        ],
        compiler_params=pltpu.CompilerParams(
            dimension_semantics=("arbitrary",),
            vmem_limit_bytes=100 * 1024 * 1024),
    )(x_tbh, w0ih, w0hh, w1ih, w1hh, b0, b1, wfc, bfc)


@functools.partial(jax.jit, static_argnames=("time_block",))
def _forward(embedding, lstm0_w_ih, lstm0_w_hh, lstm0_b_ih, lstm0_b_hh,
             lstm1_w_ih, lstm1_w_hh, lstm1_b_ih, lstm1_b_hh,
             fc_w, fc_b, tokens, time_block=16):
    H4 = 4 * embedding.shape[1]
    x = jnp.take(embedding, tokens.T, axis=0)
    b0 = (lstm0_b_ih + lstm0_b_hh).reshape(1, H4)
    b1 = (lstm1_b_ih + lstm1_b_hh).reshape(1, H4)
    out = _lstm_net(x, lstm0_w_ih, lstm0_w_hh, lstm1_w_ih, lstm1_w_hh,
                    b0, b1, jnp.transpose(fc_w), fc_b.reshape(1, -1),
                    time_block=time_block)
    return out.reshape(-1, 1)


def kernel(embedding, lstm0_w_ih, lstm0_w_hh, lstm0_b_ih, lstm0_b_hh,
           lstm1_w_ih, lstm1_w_hh, lstm1_b_ih, lstm1_b_hh,
           fc_w, fc_b, tokens):
    return _forward(embedding, lstm0_w_ih, lstm0_w_hh, lstm0_b_ih, lstm0_b_hh,
                    lstm1_w_ih, lstm1_w_hh, lstm1_b_ih, lstm1_b_hh,
                    fc_w, fc_b, tokens)
